# baseline (speedup 1.0000x reference)
import functools

import jax
import jax.numpy as jnp
from jax import lax
from jax.experimental import pallas as pl
from jax.experimental.pallas import tpu as pltpu
from jax.experimental.pallas import tpu_sc as plsc

N = 1000
P = 4
C = 16
D = 512
K = 8
S = P + C + K + 1
V = 49408
EOT_ID = V - 1

LANES = 16
VPR = D // LANES
OB = 2 * LANES

NC = 2
NS = 16
NW = NC * NS
CPW = (N + NW - 1) // NW


def _vrsqrt(x):
    i = lax.bitcast_convert_type(x, jnp.int32)
    i = jnp.int32(0x5F3759DF) - lax.shift_right_logical(i, 1)
    y = lax.bitcast_convert_type(i, jnp.float32)
    for _ in range(2):
        y = y * (jnp.float32(1.5) - jnp.float32(0.5) * x * y * y)
    return y


_IOTA = None


_GDN = lax.GatherDimensionNumbers(offset_dims=(), collapsed_slice_dims=(0,),
                                 start_index_map=(0,))


def _shuffle(v, idx):
    return lax.gather(v, idx[:, None], dimension_numbers=_GDN,
                      slice_sizes=(1,),
                      mode=lax.GatherScatterMode.PROMISE_IN_BOUNDS)


def _lane_sum(v):
    iota = lax.iota(jnp.int32, LANES)
    for m in (8, 4, 2, 1):
        v = v + _shuffle(v, jnp.bitwise_xor(iota, jnp.int32(m)))
    return v


def _row_scale(src, sr):
    vs = [src[sr, pl.ds(j * LANES, LANES)] for j in range(VPR)]
    accs = [vs[j] * vs[j] for j in range(4)]
    for j in range(4, VPR):
        accs[j % 4] = accs[j % 4] + vs[j] * vs[j]
    ssq = _lane_sum((accs[0] + accs[1]) + (accs[2] + accs[3]))
    ssq = jnp.maximum(ssq, jnp.float32(1e-24))
    return _vrsqrt(ssq)


def _row_apply(src, sr, dst, dr, scale):
    for j in range(VPR):
        dst[dr, pl.ds(j * LANES, LANES)] = \
            src[sr, pl.ds(j * LANES, LANES)] * scale


def _norm_seg(src, dst, nrows, dst_base):
    s0 = _row_scale(src, 0)

    def body(r, sc):
        sc_r = _row_scale(src, r)
        _row_apply(src, r - 1, dst, dst_base + r - 1, sc)
        return sc_r

    s_last = lax.fori_loop(1, nrows, body, s0)
    _row_apply(src, nrows - 1, dst, dst_base + nrows - 1, s_last)


def _norm_move(src, sr, dst, dr):
    vs = [src[sr, pl.ds(j * LANES, LANES)] for j in range(VPR)]
    accs = [vs[j] * vs[j] for j in range(4)]
    for j in range(4, VPR):
        accs[j % 4] = accs[j % 4] + vs[j] * vs[j]
    ssq = _lane_sum((accs[0] + accs[1]) + (accs[2] + accs[3]))
    ssq = jnp.maximum(ssq, jnp.float32(1e-24))
    scale = _vrsqrt(ssq)
    for j in range(VPR):
        dst[dr, pl.ds(j * LANES, LANES)] = vs[j] * scale


@functools.partial(
    pl.kernel,
    mesh=plsc.VectorSubcoreMesh(core_axis_name="c", subcore_axis_name="s"),
    out_type=jax.ShapeDtypeStruct((S * N, D), jnp.float32),
    scratch_types=[
        pltpu.VMEM((OB, D), jnp.float32),
        pltpu.VMEM((OB, D), jnp.float32),
        pltpu.VMEM((2, OB), jnp.int32),
        pltpu.VMEM((P, D), jnp.float32),
        pltpu.VMEM((P, D), jnp.float32),
        pltpu.VMEM((C, D), jnp.float32),
        pltpu.VMEM((C, D), jnp.float32),
        pltpu.VMEM((K, D), jnp.float32),
        pltpu.VMEM((K, D), jnp.float32),
        pltpu.VMEM((CPW * K,), jnp.int32),
        pltpu.VMEM((LANES,), jnp.int32),
        pltpu.SemaphoreType.DMA,
        pltpu.SemaphoreType.DMA,
    ],
)
def _sc_compose(prefix_hbm, context_hbm, table_hbm, ids_hbm, out_hbm,
                obuf0, obuf1, oidx, pref0, pref1, ctx0, ctx1, gat0, gat1,
                ids_v, eidx, sem_in, sem_out):
    cid = lax.axis_index("c")
    sid = lax.axis_index("s")
    wid = sid * NC + cid
    base = wid * CPW
    cnt = jnp.minimum(jnp.int32(CPW), jnp.int32(N) - base)
    base_c = jnp.minimum(base, jnp.int32(N - CPW))
    off = base - base_c
    pltpu.sync_copy(ids_hbm.at[pl.ds(base_c * K, CPW * K)], ids_v)

    obuf = (obuf0, obuf1)
    pref = (pref0, pref1)
    ctx = (ctx0, ctx1)
    gat = (gat0, gat1)

    eidx[pl.ds(0, LANES)] = jnp.full((LANES,), EOT_ID, dtype=jnp.int32)
    pltpu.async_copy(table_hbm.at[eidx.at[pl.ds(0, K)]], gat0, sem_in).wait()
    for r in range(S - 1, OB):
        _norm_move(gat0, 0, obuf0, r)
        _norm_move(gat0, 0, obuf1, r)

    def issue_in(i, b):
        @pl.when(i < cnt)
        def _():
            n = base + i
            io = pl.multiple_of((off + i) * K, 8)
            pltpu.async_copy(prefix_hbm.at[n], pref[b], sem_in)
            pltpu.async_copy(context_hbm.at[n], ctx[b], sem_in)
            pltpu.async_copy(table_hbm.at[ids_v.at[pl.ds(io, K)]],
                             gat[b], sem_in)

    def wait_in(i, b):
        @pl.when(i < cnt)
        def _():
            pltpu.make_async_copy(prefix_hbm.at[0], pref[b], sem_in).wait()
            pltpu.make_async_copy(context_hbm.at[0], ctx[b], sem_in).wait()
            pltpu.make_async_copy(table_hbm.at[pl.ds(0, K)], gat[b],
                                  sem_in).wait()

    def wait_out(b):
        pltpu.make_async_copy(obuf[b], out_hbm.at[oidx.at[b]], sem_out).wait()

    def compute(i, b):
        @pl.when(i < cnt)
        def _():
            _norm_seg(pref[b], obuf[b], P, 0)
            _norm_seg(ctx[b], obuf[b], C, P)
            _norm_seg(gat[b], obuf[b], K, P + C)
            n = base + i
            iota = lax.iota(jnp.int32, LANES)
            oidx[b, pl.ds(0, LANES)] = iota * N + n
            hi = jnp.minimum(iota + LANES, jnp.int32(S - 1))
            oidx[b, pl.ds(LANES, LANES)] = hi * N + n
            pltpu.async_copy(obuf[b], out_hbm.at[oidx.at[b]], sem_out)

    issue_in(jnp.int32(0), 0)

    def pair(g2, carry):
        for b in range(2):
            i = 2 * g2 + b
            wait_in(i, b)

            @pl.when(jnp.logical_and(i >= 1, i + 1 < cnt))
            def _():
                wait_out(1 - b)

            issue_in(i + 1, 1 - b)
            compute(i, b)
        return carry

    lax.fori_loop(0, CPW // 2, pair, None)
    wait_out(0)
    wait_out(1)


def kernel(prefix, context, table, class_ids):
    ids = class_ids.astype(jnp.int32).reshape(N * K)
    flat = _sc_compose(prefix, context, table, ids)
    seq = flat.reshape(S, N, D).transpose(1, 0, 2)
    mask = jnp.ones((N, S), dtype=bool)
    return seq, mask


# speedup vs baseline: 1.7647x; 1.7647x over previous
import functools

import jax
import jax.numpy as jnp
from jax import lax
from jax.experimental import pallas as pl
from jax.experimental.pallas import tpu as pltpu
from jax.experimental.pallas import tpu_sc as plsc

N = 1000
P = 4
C = 16
D = 512
K = 8
S = P + C + K + 1
V = 49408
EOT_ID = V - 1

LANES = 16
VPR = D // LANES
OB = 2 * LANES

NC = 2
NS = 16
NW = NC * NS
CPW = (N + NW - 1) // NW


def _vrsqrt(x):
    i = lax.bitcast_convert_type(x, jnp.int32)
    i = jnp.int32(0x5F3759DF) - lax.shift_right_logical(i, 1)
    y = lax.bitcast_convert_type(i, jnp.float32)
    for _ in range(2):
        y = y * (jnp.float32(1.5) - jnp.float32(0.5) * x * y * y)
    return y


_IOTA = None


_GDN = lax.GatherDimensionNumbers(offset_dims=(), collapsed_slice_dims=(0,),
                                 start_index_map=(0,))


def _shuffle(v, idx):
    return lax.gather(v, idx[:, None], dimension_numbers=_GDN,
                      slice_sizes=(1,),
                      mode=lax.GatherScatterMode.PROMISE_IN_BOUNDS)


def _lane_sum(v):
    iota = lax.iota(jnp.int32, LANES)
    for m in (8, 4, 2, 1):
        v = v + _shuffle(v, jnp.bitwise_xor(iota, jnp.int32(m)))
    return v


def _norm_move(src, sr, dst, dr):
    vs = [src[sr, pl.ds(j * LANES, LANES)] for j in range(VPR)]
    accs = [vs[j] * vs[j] for j in range(4)]
    for j in range(4, VPR):
        accs[j % 4] = accs[j % 4] + vs[j] * vs[j]
    ssq = _lane_sum((accs[0] + accs[1]) + (accs[2] + accs[3]))
    ssq = jnp.maximum(ssq, jnp.float32(1e-24))
    scale = _vrsqrt(ssq)
    for j in range(VPR):
        dst[dr, pl.ds(j * LANES, LANES)] = vs[j] * scale


@functools.partial(
    pl.kernel,
    mesh=plsc.VectorSubcoreMesh(core_axis_name="c", subcore_axis_name="s"),
    out_type=jax.ShapeDtypeStruct((S * N, D), jnp.float32),
    scratch_types=[
        pltpu.VMEM((OB, D), jnp.float32),
        pltpu.VMEM((OB, D), jnp.float32),
        pltpu.VMEM((2, OB), jnp.int32),
        pltpu.VMEM((P, D), jnp.float32),
        pltpu.VMEM((P, D), jnp.float32),
        pltpu.VMEM((C, D), jnp.float32),
        pltpu.VMEM((C, D), jnp.float32),
        pltpu.VMEM((K, D), jnp.float32),
        pltpu.VMEM((K, D), jnp.float32),
        pltpu.VMEM((CPW * K,), jnp.int32),
        pltpu.VMEM((LANES,), jnp.int32),
        pltpu.SemaphoreType.DMA,
        pltpu.SemaphoreType.DMA,
    ],
)
def _sc_compose(prefix_hbm, context_hbm, table_hbm, ids_hbm, out_hbm,
                obuf0, obuf1, oidx, pref0, pref1, ctx0, ctx1, gat0, gat1,
                ids_v, eidx, sem_in, sem_out):
    cid = lax.axis_index("c")
    sid = lax.axis_index("s")
    wid = sid * NC + cid
    base = wid * CPW
    cnt = jnp.minimum(jnp.int32(CPW), jnp.int32(N) - base)
    base_c = jnp.minimum(base, jnp.int32(N - CPW))
    off = base - base_c
    pltpu.sync_copy(ids_hbm.at[pl.ds(base_c * K, CPW * K)], ids_v)

    obuf = (obuf0, obuf1)
    pref = (pref0, pref1)
    ctx = (ctx0, ctx1)
    gat = (gat0, gat1)

    eidx[pl.ds(0, LANES)] = jnp.full((LANES,), EOT_ID, dtype=jnp.int32)
    pltpu.async_copy(table_hbm.at[eidx.at[pl.ds(0, K)]], gat0, sem_in).wait()
    for r in range(S - 1, OB):
        _norm_move(gat0, 0, obuf0, r)
        _norm_move(gat0, 0, obuf1, r)

    def issue_in(i, b):
        @pl.when(i < cnt)
        def _():
            n = base + i
            io = pl.multiple_of((off + i) * K, 8)
            pltpu.async_copy(prefix_hbm.at[n], pref[b], sem_in)
            pltpu.async_copy(context_hbm.at[n], ctx[b], sem_in)
            pltpu.async_copy(table_hbm.at[ids_v.at[pl.ds(io, K)]],
                             gat[b], sem_in)

    def wait_in(i, b):
        @pl.when(i < cnt)
        def _():
            pltpu.make_async_copy(prefix_hbm.at[0], pref[b], sem_in).wait()
            pltpu.make_async_copy(context_hbm.at[0], ctx[b], sem_in).wait()
            pltpu.make_async_copy(table_hbm.at[pl.ds(0, K)], gat[b],
                                  sem_in).wait()

    def wait_out(b):
        pltpu.make_async_copy(obuf[b], out_hbm.at[oidx.at[b]], sem_out).wait()

    def compute(i, b):
        @pl.when(i < cnt)
        def _():
            def norm_pref(r, c2):
                _norm_move(pref[b], r, obuf[b], r)
                return c2

            def norm_ctx(r, c2):
                _norm_move(ctx[b], r, obuf[b], P + r)
                return c2

            def norm_gat(r, c2):
                _norm_move(gat[b], r, obuf[b], P + C + r)
                return c2

            lax.fori_loop(0, P, norm_pref, None)
            lax.fori_loop(0, C, norm_ctx, None)
            lax.fori_loop(0, K, norm_gat, None)
            n = base + i
            iota = lax.iota(jnp.int32, LANES)
            oidx[b, pl.ds(0, LANES)] = iota * N + n
            hi = jnp.minimum(iota + LANES, jnp.int32(S - 1))
            oidx[b, pl.ds(LANES, LANES)] = hi * N + n
            pltpu.async_copy(obuf[b], out_hbm.at[oidx.at[b]], sem_out)

    issue_in(jnp.int32(0), 0)

    def pair(g2, carry):
        for b in range(2):
            i = 2 * g2 + b
            wait_in(i, b)

            @pl.when(jnp.logical_and(i >= 1, i + 1 < cnt))
            def _():
                wait_out(1 - b)

            issue_in(i + 1, 1 - b)
            compute(i, b)
        return carry

    lax.fori_loop(0, CPW // 2, pair, None)
    wait_out(0)
    wait_out(1)


def kernel(prefix, context, table, class_ids):
    ids = class_ids.astype(jnp.int32).reshape(N * K)
    flat = _sc_compose(prefix, context, table, ids)
    seq = flat.reshape(S, N, D).transpose(1, 0, 2)
    mask = jnp.ones((N, S), dtype=bool)
    return seq, mask


# speedup vs baseline: 1.8205x; 1.0316x over previous
import functools

import jax
import jax.numpy as jnp
from jax import lax
from jax.experimental import pallas as pl
from jax.experimental.pallas import tpu as pltpu
from jax.experimental.pallas import tpu_sc as plsc

N = 1000
P = 4
C = 16
D = 512
K = 8
S = P + C + K + 1
V = 49408
EOT_ID = V - 1

LANES = 16
VPR = D // LANES
OB = 2 * LANES

NC = 2
NS = 16
NW = NC * NS
CPW = (N + NW - 1) // NW


def _vrsqrt(x):
    i = lax.bitcast_convert_type(x, jnp.int32)
    i = jnp.int32(0x5F3759DF) - lax.shift_right_logical(i, 1)
    y = lax.bitcast_convert_type(i, jnp.float32)
    for _ in range(2):
        y = y * (jnp.float32(1.5) - jnp.float32(0.5) * x * y * y)
    return y


_IOTA = None


_GDN = lax.GatherDimensionNumbers(offset_dims=(), collapsed_slice_dims=(0,),
                                 start_index_map=(0,))


def _shuffle(v, idx):
    return lax.gather(v, idx[:, None], dimension_numbers=_GDN,
                      slice_sizes=(1,),
                      mode=lax.GatherScatterMode.PROMISE_IN_BOUNDS)


def _lane_sum(v):
    iota = lax.iota(jnp.int32, LANES)
    for m in (8, 4, 2, 1):
        v = v + _shuffle(v, jnp.bitwise_xor(iota, jnp.int32(m)))
    return v


def _row_scale_from(vs):
    accs = [vs[j] * vs[j] for j in range(4)]
    for j in range(4, VPR):
        accs[j % 4] = accs[j % 4] + vs[j] * vs[j]
    ssq = _lane_sum((accs[0] + accs[1]) + (accs[2] + accs[3]))
    ssq = jnp.maximum(ssq, jnp.float32(1e-24))
    return _vrsqrt(ssq)


def _norm_seg(src, dst, nrows, dst_base):
    v0 = [src[0, pl.ds(j * LANES, LANES)] for j in range(VPR)]
    c0 = (*v0, _row_scale_from(v0))

    def body(r, carry):
        vp, scp = carry[:VPR], carry[VPR]
        vr = []
        for j in range(VPR):
            vr.append(src[r, pl.ds(j * LANES, LANES)])
            dst[dst_base + r - 1, pl.ds(j * LANES, LANES)] = vp[j] * scp
        return (*vr, _row_scale_from(vr))

    carry = lax.fori_loop(1, nrows, body, c0)
    vl, scl = carry[:VPR], carry[VPR]
    for j in range(VPR):
        dst[dst_base + nrows - 1, pl.ds(j * LANES, LANES)] = vl[j] * scl


def _norm_move(src, sr, dst, dr):
    vs = [src[sr, pl.ds(j * LANES, LANES)] for j in range(VPR)]
    accs = [vs[j] * vs[j] for j in range(4)]
    for j in range(4, VPR):
        accs[j % 4] = accs[j % 4] + vs[j] * vs[j]
    ssq = _lane_sum((accs[0] + accs[1]) + (accs[2] + accs[3]))
    ssq = jnp.maximum(ssq, jnp.float32(1e-24))
    scale = _vrsqrt(ssq)
    for j in range(VPR):
        dst[dr, pl.ds(j * LANES, LANES)] = vs[j] * scale


@functools.partial(
    pl.kernel,
    mesh=plsc.VectorSubcoreMesh(core_axis_name="c", subcore_axis_name="s"),
    out_type=jax.ShapeDtypeStruct((S * N, D), jnp.float32),
    scratch_types=[
        pltpu.VMEM((OB, D), jnp.float32),
        pltpu.VMEM((OB, D), jnp.float32),
        pltpu.VMEM((2, OB), jnp.int32),
        pltpu.VMEM((P, D), jnp.float32),
        pltpu.VMEM((P, D), jnp.float32),
        pltpu.VMEM((C, D), jnp.float32),
        pltpu.VMEM((C, D), jnp.float32),
        pltpu.VMEM((K, D), jnp.float32),
        pltpu.VMEM((K, D), jnp.float32),
        pltpu.VMEM((CPW * K,), jnp.int32),
        pltpu.VMEM((LANES,), jnp.int32),
        pltpu.SemaphoreType.DMA,
        pltpu.SemaphoreType.DMA,
    ],
)
def _sc_compose(prefix_hbm, context_hbm, table_hbm, ids_hbm, out_hbm,
                obuf0, obuf1, oidx, pref0, pref1, ctx0, ctx1, gat0, gat1,
                ids_v, eidx, sem_in, sem_out):
    cid = lax.axis_index("c")
    sid = lax.axis_index("s")
    wid = sid * NC + cid
    base = wid * CPW
    cnt = jnp.minimum(jnp.int32(CPW), jnp.int32(N) - base)
    base_c = jnp.minimum(base, jnp.int32(N - CPW))
    off = base - base_c
    pltpu.sync_copy(ids_hbm.at[pl.ds(base_c * K, CPW * K)], ids_v)

    obuf = (obuf0, obuf1)
    pref = (pref0, pref1)
    ctx = (ctx0, ctx1)
    gat = (gat0, gat1)

    eidx[pl.ds(0, LANES)] = jnp.full((LANES,), EOT_ID, dtype=jnp.int32)
    pltpu.async_copy(table_hbm.at[eidx.at[pl.ds(0, K)]], gat0, sem_in).wait()
    for r in range(S - 1, OB):
        _norm_move(gat0, 0, obuf0, r)
        _norm_move(gat0, 0, obuf1, r)

    def issue_in(i, b):
        @pl.when(i < cnt)
        def _():
            n = base + i
            io = pl.multiple_of((off + i) * K, 8)
            pltpu.async_copy(prefix_hbm.at[n], pref[b], sem_in)
            pltpu.async_copy(context_hbm.at[n], ctx[b], sem_in)
            pltpu.async_copy(table_hbm.at[ids_v.at[pl.ds(io, K)]],
                             gat[b], sem_in)

    def wait_in(i, b):
        @pl.when(i < cnt)
        def _():
            pltpu.make_async_copy(prefix_hbm.at[0], pref[b], sem_in).wait()
            pltpu.make_async_copy(context_hbm.at[0], ctx[b], sem_in).wait()
            pltpu.make_async_copy(table_hbm.at[pl.ds(0, K)], gat[b],
                                  sem_in).wait()

    def wait_out(b):
        pltpu.make_async_copy(obuf[b], out_hbm.at[oidx.at[b]], sem_out).wait()

    def compute(i, b):
        @pl.when(i < cnt)
        def _():
            _norm_seg(pref[b], obuf[b], P, 0)
            _norm_seg(ctx[b], obuf[b], C, P)
            _norm_seg(gat[b], obuf[b], K, P + C)
            n = base + i
            iota = lax.iota(jnp.int32, LANES)
            oidx[b, pl.ds(0, LANES)] = iota * N + n
            hi = jnp.minimum(iota + LANES, jnp.int32(S - 1))
            oidx[b, pl.ds(LANES, LANES)] = hi * N + n
            pltpu.async_copy(obuf[b], out_hbm.at[oidx.at[b]], sem_out)

    issue_in(jnp.int32(0), 0)

    def pair(g2, carry):
        for b in range(2):
            i = 2 * g2 + b
            wait_in(i, b)

            @pl.when(jnp.logical_and(i >= 1, i + 1 < cnt))
            def _():
                wait_out(1 - b)

            issue_in(i + 1, 1 - b)
            compute(i, b)
        return carry

    lax.fori_loop(0, CPW // 2, pair, None)
    wait_out(0)
    wait_out(1)


def kernel(prefix, context, table, class_ids):
    ids = class_ids.astype(jnp.int32).reshape(N * K)
    flat = _sc_compose(prefix, context, table, ids)
    seq = flat.reshape(S, N, D).transpose(1, 0, 2)
    mask = jnp.ones((N, S), dtype=bool)
    return seq, mask


# speedup vs baseline: 1.9830x; 1.0893x over previous
import functools

import jax
import jax.numpy as jnp
from jax import lax
from jax.experimental import pallas as pl
from jax.experimental.pallas import tpu as pltpu
from jax.experimental.pallas import tpu_sc as plsc

N = 1000
P = 4
C = 16
D = 512
K = 8
S = P + C + K + 1
V = 49408
EOT_ID = V - 1

LANES = 16
VPR = D // LANES
OB = 2 * LANES

NC = 2
NS = 16
NW = NC * NS
CPW = (N + NW - 1) // NW


def _vrsqrt(x):
    i = lax.bitcast_convert_type(x, jnp.int32)
    i = jnp.int32(0x5F3759DF) - lax.shift_right_logical(i, 1)
    y = lax.bitcast_convert_type(i, jnp.float32)
    for _ in range(2):
        y = y * (jnp.float32(1.5) - jnp.float32(0.5) * x * y * y)
    return y


_IOTA = None


_GDN = lax.GatherDimensionNumbers(offset_dims=(), collapsed_slice_dims=(0,),
                                 start_index_map=(0,))


def _shuffle(v, idx):
    return lax.gather(v, idx[:, None], dimension_numbers=_GDN,
                      slice_sizes=(1,),
                      mode=lax.GatherScatterMode.PROMISE_IN_BOUNDS)


def _lane_sum(v):
    iota = lax.iota(jnp.int32, LANES)
    for m in (8, 4, 2, 1):
        v = v + _shuffle(v, jnp.bitwise_xor(iota, jnp.int32(m)))
    return v


def _row_scale_from(vs):
    accs = [vs[j] * vs[j] for j in range(4)]
    for j in range(4, VPR):
        accs[j % 4] = accs[j % 4] + vs[j] * vs[j]
    ssq = _lane_sum((accs[0] + accs[1]) + (accs[2] + accs[3]))
    ssq = jnp.maximum(ssq, jnp.float32(1e-24))
    return _vrsqrt(ssq)


def _seg_start(src):
    v0 = [src[0, pl.ds(j * LANES, LANES)] for j in range(VPR)]
    return (*v0, _row_scale_from(v0))


def _seg_run(src, dst, lo, nrows, dst_base, carry):
    def body(r, c):
        vp, scp = c[:VPR], c[VPR]
        vr = []
        for j in range(VPR):
            vr.append(src[r, pl.ds(j * LANES, LANES)])
            dst[dst_base + r - 1, pl.ds(j * LANES, LANES)] = vp[j] * scp
        return (*vr, _row_scale_from(vr))

    return lax.fori_loop(lo, nrows, body, carry)


def _seg_end(dst, dr, carry):
    vl, scl = carry[:VPR], carry[VPR]
    for j in range(VPR):
        dst[dr, pl.ds(j * LANES, LANES)] = vl[j] * scl


def _norm_move(src, sr, dst, dr):
    vs = [src[sr, pl.ds(j * LANES, LANES)] for j in range(VPR)]
    accs = [vs[j] * vs[j] for j in range(4)]
    for j in range(4, VPR):
        accs[j % 4] = accs[j % 4] + vs[j] * vs[j]
    ssq = _lane_sum((accs[0] + accs[1]) + (accs[2] + accs[3]))
    ssq = jnp.maximum(ssq, jnp.float32(1e-24))
    scale = _vrsqrt(ssq)
    for j in range(VPR):
        dst[dr, pl.ds(j * LANES, LANES)] = vs[j] * scale


@functools.partial(
    pl.kernel,
    mesh=plsc.VectorSubcoreMesh(core_axis_name="c", subcore_axis_name="s"),
    out_type=jax.ShapeDtypeStruct((S * N, D), jnp.float32),
    scratch_types=[
        pltpu.VMEM((OB, D), jnp.float32),
        pltpu.VMEM((OB, D), jnp.float32),
        pltpu.VMEM((2, OB), jnp.int32),
        pltpu.VMEM((P, D), jnp.float32),
        pltpu.VMEM((P, D), jnp.float32),
        pltpu.VMEM((C, D), jnp.float32),
        pltpu.VMEM((C, D), jnp.float32),
        pltpu.VMEM((K, D), jnp.float32),
        pltpu.VMEM((K, D), jnp.float32),
        pltpu.VMEM((CPW * K,), jnp.int32),
        pltpu.VMEM((LANES,), jnp.int32),
        pltpu.SemaphoreType.DMA,
        pltpu.SemaphoreType.DMA,
    ],
)
def _sc_compose(prefix_hbm, context_hbm, table_hbm, ids_hbm, out_hbm,
                obuf0, obuf1, oidx, pref0, pref1, ctx0, ctx1, gat0, gat1,
                ids_v, eidx, sem_in, sem_out):
    cid = lax.axis_index("c")
    sid = lax.axis_index("s")
    wid = sid * NC + cid
    base = wid * CPW
    cnt = jnp.minimum(jnp.int32(CPW), jnp.int32(N) - base)
    base_c = jnp.minimum(base, jnp.int32(N - CPW))
    off = base - base_c
    pltpu.sync_copy(ids_hbm.at[pl.ds(base_c * K, CPW * K)], ids_v)

    obuf = (obuf0, obuf1)
    pref = (pref0, pref1)
    ctx = (ctx0, ctx1)
    gat = (gat0, gat1)

    eidx[pl.ds(0, LANES)] = jnp.full((LANES,), EOT_ID, dtype=jnp.int32)
    pltpu.async_copy(table_hbm.at[eidx.at[pl.ds(0, K)]], gat0, sem_in).wait()
    for r in range(S - 1, OB):
        _norm_move(gat0, 0, obuf0, r)
        _norm_move(gat0, 0, obuf1, r)

    def issue_in(i, b):
        @pl.when(i < cnt)
        def _():
            n = base + i
            io = pl.multiple_of((off + i) * K, 8)
            pltpu.async_copy(prefix_hbm.at[n], pref[b], sem_in)
            pltpu.async_copy(context_hbm.at[n], ctx[b], sem_in)
            pltpu.async_copy(table_hbm.at[ids_v.at[pl.ds(io, K)]],
                             gat[b], sem_in)

    def wait_in(i, b):
        @pl.when(i < cnt)
        def _():
            pltpu.make_async_copy(prefix_hbm.at[0], pref[b], sem_in).wait()
            pltpu.make_async_copy(context_hbm.at[0], ctx[b], sem_in).wait()
            pltpu.make_async_copy(table_hbm.at[pl.ds(0, K)], gat[b],
                                  sem_in).wait()

    def wait_out(b):
        pltpu.make_async_copy(obuf[b], out_hbm.at[oidx.at[b]], sem_out).wait()

    def compute(i, b):
        @pl.when(i < cnt)
        def _():
            cch = _seg_start(pref[b])
            cch = _seg_run(pref[b], obuf[b], 1, P, 0, cch)
            cch = _seg_run(ctx[b], obuf[b], 0, C, P, cch)
            cch = _seg_run(gat[b], obuf[b], 0, K, P + C, cch)
            _seg_end(obuf[b], P + C + K - 1, cch)
            n = base + i
            iota = lax.iota(jnp.int32, LANES)
            oidx[b, pl.ds(0, LANES)] = iota * N + n
            hi = jnp.minimum(iota + LANES, jnp.int32(S - 1))
            oidx[b, pl.ds(LANES, LANES)] = hi * N + n
            pltpu.async_copy(obuf[b], out_hbm.at[oidx.at[b]], sem_out)

    issue_in(jnp.int32(0), 0)

    def pair(g2, carry):
        for b in range(2):
            i = 2 * g2 + b
            wait_in(i, b)

            @pl.when(jnp.logical_and(i >= 1, i + 1 < cnt))
            def _():
                wait_out(1 - b)

            issue_in(i + 1, 1 - b)
            compute(i, b)
        return carry

    lax.fori_loop(0, CPW // 2, pair, None)
    wait_out(0)
    wait_out(1)


def kernel(prefix, context, table, class_ids):
    ids = class_ids.astype(jnp.int32).reshape(N * K)
    flat = _sc_compose(prefix, context, table, ids)
    seq = flat.reshape(S, N, D).transpose(1, 0, 2)
    mask = jnp.ones((N, S), dtype=bool)
    return seq, mask
